# baseline (device time: 43013 ns/iter reference)
import jax
import jax.numpy as jnp
from jax import lax
from jax.experimental import pallas as pl
from jax.experimental.pallas import tpu as pltpu

N_DEV = 4
N_RDMA = 18


def kernel(x, Win0, Wout0, Win1, Wout1, Win2, Wout2):
    m_per, d = x.shape
    _, h_per = Win0.shape

    def body(x_ref, win0, wout0, win1, wout1, win2, wout2, out_ref,
             W0, W1, W2, V0, V1, V2, send_sems, recv_sems):
        me = lax.axis_index("i")
        ypart = me ^ 1
        xpart = 3 - me

        sem = iter(range(N_RDMA))

        def start(*quads):
            rdmas = []
            for src, dst, b, partner in quads:
                i = next(sem)
                r = pltpu.make_async_remote_copy(
                    src_ref=src.at[b],
                    dst_ref=dst.at[b],
                    send_sem=send_sems.at[i],
                    recv_sem=recv_sems.at[i],
                    device_id=(partner,),
                    device_id_type=pl.DeviceIdType.MESH,
                )
                r.start()
                rdmas.append(r)
            return rdmas

        def wait(rdmas):
            for r in rdmas:
                r.wait()

        for ref, src_ref in (
            (W0, win0), (W1, win1), (W2, win2),
            (V0, wout0), (V1, wout1), (V2, wout2),
        ):
            ref[pl.ds(me, 1)] = src_ref[...].astype(jnp.bfloat16)[None]
        xl = x_ref[...].astype(jnp.bfloat16)

        barrier = pltpu.get_barrier_semaphore()
        for nbr in (ypart, xpart):
            pl.semaphore_signal(
                barrier, inc=1,
                device_id=(nbr,), device_id_type=pl.DeviceIdType.MESH,
            )
        pl.semaphore_wait(barrier, 2)

        r1 = [
            start((W, W, me, ypart), (V, V, me, xpart))
            for W, V in ((W0, V0), (W1, V1), (W2, V2))
        ]
        r2 = []
        for (W, V), r in zip(((W0, V0), (W1, V1), (W2, V2)), r1):
            wait(r)
            r2.append(start(
                (W, W, me, xpart), (W, W, ypart, xpart),
                (V, V, me, ypart), (V, V, xpart, ypart),
            ))

        for l, (W, V) in enumerate(((W0, V0), (W1, V1), (W2, V2))):
            wait(r2[l])
            h3 = lax.dot_general(
                xl, W[...],
                dimension_numbers=(((1,), (1,)), ((), ())),
                preferred_element_type=jnp.float32,
            )
            hb = (
                jnp.maximum(h3, 0.0)
                .astype(jnp.bfloat16)
                .reshape(m_per, N_DEV * h_per)
            )
            acc = jnp.dot(
                hb, V[...].reshape(N_DEV * h_per, d),
                preferred_element_type=jnp.float32,
            )
            if l < 2:
                xl = acc.astype(jnp.bfloat16)
            else:
                out_ref[...] = acc

    return pl.pallas_call(
        body,
        out_shape=jax.ShapeDtypeStruct((m_per, d), jnp.float32),
        in_specs=[pl.BlockSpec(memory_space=pltpu.VMEM)] * 7,
        out_specs=pl.BlockSpec(memory_space=pltpu.VMEM),
        scratch_shapes=[
            pltpu.VMEM((N_DEV, d, h_per), jnp.bfloat16),
            pltpu.VMEM((N_DEV, d, h_per), jnp.bfloat16),
            pltpu.VMEM((N_DEV, d, h_per), jnp.bfloat16),
            pltpu.VMEM((N_DEV, h_per, d), jnp.bfloat16),
            pltpu.VMEM((N_DEV, h_per, d), jnp.bfloat16),
            pltpu.VMEM((N_DEV, h_per, d), jnp.bfloat16),
            pltpu.SemaphoreType.DMA((N_RDMA,)),
            pltpu.SemaphoreType.DMA((N_RDMA,)),
        ],
        compiler_params=pltpu.CompilerParams(collective_id=0),
    )(x, Win0, Wout0, Win1, Wout1, Win2, Wout2)
